# Initial kernel scaffold; baseline (speedup 1.0000x reference)
import jax
import jax.numpy as jnp
from jax.experimental import pallas as pl


def kernel(class_logits, box_regression, proposals):
    raise NotImplementedError("write your pallas kernel here")


# speedup vs baseline: 236.8801x; 236.8801x over previous
import functools
import math

import jax
import jax.numpy as jnp
from jax import lax
from jax.experimental import pallas as pl
from jax.experimental.pallas import tpu as pltpu
from jax.experimental.pallas import tpu_sc as plsc

N = 5000
NPAD = 5120
NCLS = 81
NC = 80
CAP = 112
DETS = 100
SCORE_THRESH = 0.05
NMS_THRESH = 0.5
IMG_W = 1024.0
IMG_H = 1024.0
BBOX_XFORM_CLIP = float(math.log(1000.0 / 16.0))
NEG = -1e30
L = 16
BLK = 1280


def _dense_body(lg_ref, dx_ref, dy_ref, dw_ref, dh_ref, pr_ref,
                s_ref, x1_ref, y1_ref, x2_ref, y2_ref):
    l = lg_ref[...]
    m = jnp.max(l, axis=0, keepdims=True)
    e = jnp.exp(l - m)
    den = jnp.sum(e, axis=0, keepdims=True)
    s_ref[...] = e[1:, :] / den

    px1 = pr_ref[0:1, :]
    py1 = pr_ref[1:2, :]
    px2 = pr_ref[2:3, :]
    py2 = pr_ref[3:4, :]
    w = px2 - px1 + 1.0
    h = py2 - py1 + 1.0
    cx = px1 + 0.5 * w
    cy = py1 + 0.5 * h
    dx = dx_ref[...] / 10.0
    dy = dy_ref[...] / 10.0
    dw = jnp.minimum(dw_ref[...] / 5.0, BBOX_XFORM_CLIP)
    dh = jnp.minimum(dh_ref[...] / 5.0, BBOX_XFORM_CLIP)
    pcx = dx * w + cx
    pcy = dy * h + cy
    pw = jnp.exp(dw) * w
    ph = jnp.exp(dh) * h
    x1_ref[...] = jnp.clip(pcx - 0.5 * pw, 0.0, IMG_W - 1.0)
    y1_ref[...] = jnp.clip(pcy - 0.5 * ph, 0.0, IMG_H - 1.0)
    x2_ref[...] = jnp.clip(pcx + 0.5 * pw - 1.0, 0.0, IMG_W - 1.0)
    y2_ref[...] = jnp.clip(pcy + 0.5 * ph - 1.0, 0.0, IMG_H - 1.0)


def _dense(logits_t, dxt, dyt, dwt, dht, prop_t, interpret=False):
    nblk = NPAD // BLK
    cm = pl.BlockSpec((NC, BLK), lambda j: (0, j))
    return pl.pallas_call(
        _dense_body,
        grid=(nblk,),
        in_specs=[
            pl.BlockSpec((NCLS, BLK), lambda j: (0, j)),
            cm, cm, cm, cm,
            pl.BlockSpec((8, BLK), lambda j: (0, j)),
        ],
        out_specs=[cm] * 5,
        out_shape=[jax.ShapeDtypeStruct((NC, NPAD), jnp.float32)] * 5,
        interpret=interpret,
    )(logits_t, dxt, dyt, dwt, dht, prop_t)


def _nms_body(s_hbm, x1_hbm, y1_hbm, x2_hbm, y2_hbm,
              ks_hbm, kx1_hbm, ky1_hbm, kx2_hbm, ky2_hbm,
              in_s, in_x1, in_y1, in_x2, in_y2,
              c_s, c_x1, c_y1, c_x2, c_y2, c_a,
              ko_scr, ko_x1, ko_y1, ko_x2, ko_y2):
    wid = lax.axis_index("s") * 2 + lax.axis_index("c")
    lane = lax.iota(jnp.int32, L)
    rm0 = jnp.full((L,), NEG, jnp.float32)
    ri0 = jnp.zeros((L,), jnp.int32)

    def process(cidx):
        pltpu.sync_copy(s_hbm.at[cidx], in_s)
        pltpu.sync_copy(x1_hbm.at[cidx], in_x1)
        pltpu.sync_copy(y1_hbm.at[cidx], in_y1)
        pltpu.sync_copy(x2_hbm.at[cidx], in_x2)
        pltpu.sync_copy(y2_hbm.at[cidx], in_y2)

        def comp(i, cnt):
            sl = pl.ds(i * L, L)
            v = in_s[sl]
            m = v > SCORE_THRESH
            cs = plsc.cumsum(jnp.where(m, jnp.int32(1), jnp.int32(0)))
            idx = cnt + cs - 1
            x1v = in_x1[sl]
            y1v = in_y1[sl]
            x2v = in_x2[sl]
            y2v = in_y2[sl]
            plsc.store_scatter(c_s, [idx], v, mask=m)
            plsc.store_scatter(c_x1, [idx], x1v, mask=m)
            plsc.store_scatter(c_y1, [idx], y1v, mask=m)
            plsc.store_scatter(c_x2, [idx], x2v, mask=m)
            plsc.store_scatter(c_y2, [idx], y2v, mask=m)
            area = (x2v - x1v + 1.0) * (y2v - y1v + 1.0)
            plsc.store_scatter(c_a, [idx], area, mask=m)
            return cnt + jnp.max(cs)

        v_cnt = lax.fori_loop(0, NPAD // L, comp, jnp.int32(0))
        nv = (v_cnt + L - 1) // L

        tail_idx = jnp.minimum(v_cnt + lane, NPAD - 1)
        tail_m = (v_cnt + lane) < nv * L
        plsc.store_scatter(c_s, [tail_idx], jnp.full((L,), NEG, jnp.float32),
                           mask=tail_m)

        for j in range(CAP // L):
            ko_scr[pl.ds(j * L, L)] = jnp.full((L,), NEG, jnp.float32)

        def am(i, carry):
            rm, ri = carry
            sl = pl.ds(i * L, L)
            v = c_s[sl]
            gi = i * L + lane
            upd = v > rm
            return jnp.where(upd, v, rm), jnp.where(upd, gi, ri)

        rm, ri = lax.fori_loop(0, nv, am, (rm0, ri0))
        ms = jnp.max(rm)
        cur = jnp.min(jnp.where(rm == ms, ri, jnp.int32(2**30)))

        def w_cond(carry):
            _, ms_c, _ = carry
            return ms_c > 0.0

        def w_body(carry):
            cur_c, ms_c, kcnt = carry
            curv = jnp.full((L,), cur_c)
            bx1 = plsc.load_gather(c_x1, [curv])
            by1 = plsc.load_gather(c_y1, [curv])
            bx2 = plsc.load_gather(c_x2, [curv])
            by2 = plsc.load_gather(c_y2, [curv])
            bar = plsc.load_gather(c_a, [curv])

            @pl.when(kcnt < DETS)
            def _():
                kidx = jnp.full((L,), kcnt)
                m0 = lane == 0
                plsc.store_scatter(ko_scr, [kidx], jnp.full((L,), ms_c), mask=m0)
                plsc.store_scatter(ko_x1, [kidx], bx1, mask=m0)
                plsc.store_scatter(ko_y1, [kidx], by1, mask=m0)
                plsc.store_scatter(ko_x2, [kidx], bx2, mask=m0)
                plsc.store_scatter(ko_y2, [kidx], by2, mask=m0)

            def sp(i, carry2):
                rm_i, ri_i = carry2
                sl = pl.ds(i * L, L)
                s = c_s[sl]
                x1c = c_x1[sl]
                y1c = c_y1[sl]
                x2c = c_x2[sl]
                y2c = c_y2[sl]
                ac = c_a[sl]
                xx1 = jnp.maximum(bx1, x1c)
                yy1 = jnp.maximum(by1, y1c)
                xx2 = jnp.minimum(bx2, x2c)
                yy2 = jnp.minimum(by2, y2c)
                inter = (jnp.maximum(0.0, xx2 - xx1 + 1.0)
                         * jnp.maximum(0.0, yy2 - yy1 + 1.0))
                iou = inter / (bar + ac - inter)
                gi = i * L + lane
                supp = (iou > NMS_THRESH) | (gi == cur_c)
                ns = jnp.where(supp, NEG, s)
                c_s[sl] = ns
                upd = ns > rm_i
                return jnp.where(upd, ns, rm_i), jnp.where(upd, gi, ri_i)

            rm2, ri2 = lax.fori_loop(0, nv, sp, (rm0, ri0))
            ms2 = jnp.max(rm2)
            cur2 = jnp.min(jnp.where(rm2 == ms2, ri2, jnp.int32(2**30)))
            return cur2, ms2, kcnt + 1

        lax.while_loop(w_cond, w_body, (cur, ms, jnp.int32(0)))

        pltpu.sync_copy(ko_scr, ks_hbm.at[cidx])
        pltpu.sync_copy(ko_x1, kx1_hbm.at[cidx])
        pltpu.sync_copy(ko_y1, ky1_hbm.at[cidx])
        pltpu.sync_copy(ko_x2, kx2_hbm.at[cidx])
        pltpu.sync_copy(ko_y2, ky2_hbm.at[cidx])

    process(wid)
    process(wid + 32)

    @pl.when(wid + 64 < NC)
    def _():
        process(wid + 64)


def _nms(scm, x1cm, y1cm, x2cm, y2cm):
    mesh = plsc.VectorSubcoreMesh(core_axis_name="c", subcore_axis_name="s",
                                  num_cores=2, num_subcores=16)
    f32 = jnp.float32
    out = jax.ShapeDtypeStruct((NC, CAP), f32)
    return pl.kernel(
        _nms_body,
        out_type=[out] * 5,
        mesh=mesh,
        compiler_params=pltpu.CompilerParams(needs_layout_passes=False),
        scratch_types=(
            [pltpu.VMEM((NPAD,), f32)] * 5
            + [pltpu.VMEM((NPAD,), f32)] * 6
            + [pltpu.VMEM((CAP,), f32)] * 5
        ),
    )(scm, x1cm, y1cm, x2cm, y2cm)


def _merge_body(ks, kx1, ky1, kx2, ky2, scm, x1cm, y1cm, x2cm, y2cm,
                ob_hbm, os_hbm, ol_hbm,
                ks_v, kx1_v, ky1_v, kx2_v, ky2_v,
                hs_v, pos_v, tmp16, ob_v, os_v, ol_v):
    wid = lax.axis_index("s") * 2 + lax.axis_index("c")
    lane = lax.iota(jnp.int32, L)
    zero16 = jnp.zeros((L,), jnp.int32)
    rm0 = jnp.full((L,), NEG, jnp.float32)

    @pl.when(wid == 0)
    def _():
        pltpu.sync_copy(ks, ks_v)
        pltpu.sync_copy(kx1, kx1_v)
        pltpu.sync_copy(ky1, ky1_v)
        pltpu.sync_copy(kx2, kx2_v)
        pltpu.sync_copy(ky2, ky2_v)

        def first_elem(hbm):
            pltpu.sync_copy(hbm.at[0, pl.ds(0, L)], tmp16)
            return jnp.max(plsc.load_gather(tmp16, [zero16]))

        d_scr = first_elem(scm)
        d_x1 = first_elem(x1cm)
        d_y1 = first_elem(y1cm)
        d_x2 = first_elem(x2cm)
        d_y2 = first_elem(y2cm)

        for j in range(128 // L):
            hs_v[pl.ds(j * L, L)] = jnp.full((L,), NEG, jnp.float32)
            pos_v[pl.ds(j * L, L)] = jnp.zeros((L,), jnp.int32)

        def init_head(c, _):
            cv = jnp.full((L,), c)
            v = plsc.load_gather(ks_v, [cv, zero16])
            plsc.store_scatter(hs_v, [cv], v, mask=lane == 0)
            return 0

        lax.fori_loop(0, NC, init_head, 0)

        def step(t, carry):
            lcls, lscr, lx1, ly1, lx2, ly2 = carry

            def am(j, carry2):
                rm, ri = carry2
                v = hs_v[pl.ds(j * L, L)]
                gi = j * L + lane
                upd = v > rm
                return jnp.where(upd, v, rm), jnp.where(upd, gi, ri)

            rm, ri = lax.fori_loop(0, 128 // L, am,
                                   (rm0, jnp.zeros((L,), jnp.int32)))
            ms = jnp.max(rm)
            cstar = jnp.min(jnp.where(rm == ms, ri, jnp.int32(2**30)))
            ex = ms <= jnp.float32(-1e29)
            cs_safe = jnp.where(ex, 0, cstar)
            csv = jnp.full((L,), cs_safe)
            p = jnp.max(plsc.load_gather(pos_v, [csv]))
            pv = jnp.full((L,), p)
            bx1 = jnp.max(plsc.load_gather(kx1_v, [csv, pv]))
            by1 = jnp.max(plsc.load_gather(ky1_v, [csv, pv]))
            bx2 = jnp.max(plsc.load_gather(kx2_v, [csv, pv]))
            by2 = jnp.max(plsc.load_gather(ky2_v, [csv, pv]))
            pn = jnp.minimum(p + 1, CAP - 1)
            nh = plsc.load_gather(ks_v, [csv, jnp.full((L,), pn)])
            plsc.store_scatter(pos_v, [csv], jnp.full((L,), pn), mask=lane == 0)
            plsc.store_scatter(hs_v, [csv], nh, mask=lane == 0)

            ncls = jnp.where(ex, lcls, cs_safe + 1)
            nscr = jnp.where(ex, lscr, ms)
            nx1 = jnp.where(ex, lx1, bx1)
            ny1 = jnp.where(ex, ly1, by1)
            nx2 = jnp.where(ex, lx2, bx2)
            ny2 = jnp.where(ex, ly2, by2)

            tv = jnp.full((L,), t)
            plsc.store_scatter(os_v, [tv], jnp.full((L,), nscr), mask=lane == 0)
            plsc.store_scatter(ol_v, [tv], jnp.full((L,), ncls), mask=lane == 0)
            bval = jnp.where(lane == 0, nx1,
                             jnp.where(lane == 1, ny1,
                                       jnp.where(lane == 2, nx2, ny2)))
            plsc.store_scatter(ob_v, [t * 4 + lane], bval, mask=lane < 4)
            return ncls, nscr, nx1, ny1, nx2, ny2

        lax.fori_loop(0, DETS, step,
                      (jnp.int32(1), d_scr, d_x1, d_y1, d_x2, d_y2))

        pltpu.sync_copy(ob_v, ob_hbm)
        pltpu.sync_copy(os_v, os_hbm)
        pltpu.sync_copy(ol_v, ol_hbm)


def _merge(ks, kx1, ky1, kx2, ky2, scm, x1cm, y1cm, x2cm, y2cm):
    mesh = plsc.VectorSubcoreMesh(core_axis_name="c", subcore_axis_name="s",
                                  num_cores=2, num_subcores=16)
    f32 = jnp.float32
    i32 = jnp.int32
    return pl.kernel(
        _merge_body,
        compiler_params=pltpu.CompilerParams(needs_layout_passes=False),
        out_type=[
            jax.ShapeDtypeStruct((CAP * 4,), f32),
            jax.ShapeDtypeStruct((CAP,), f32),
            jax.ShapeDtypeStruct((CAP,), i32),
        ],
        mesh=mesh,
        scratch_types=(
            [pltpu.VMEM((NC, CAP), f32)] * 5
            + [pltpu.VMEM((128,), f32), pltpu.VMEM((128,), i32),
               pltpu.VMEM((L,), f32),
               pltpu.VMEM((CAP * 4,), f32), pltpu.VMEM((CAP,), f32),
               pltpu.VMEM((CAP,), i32)]
        ),
    )(ks, kx1, ky1, kx2, ky2, scm, x1cm, y1cm, x2cm, y2cm)


def kernel(class_logits, box_regression, proposals):
    pad = NPAD - N
    lg = jnp.pad(class_logits, ((0, pad), (0, 0)))
    br = box_regression.reshape(N, NCLS, 4)
    dxt = jnp.pad(br[:, 1:, 0], ((0, pad), (0, 0))).T
    dyt = jnp.pad(br[:, 1:, 1], ((0, pad), (0, 0))).T
    dwt = jnp.pad(br[:, 1:, 2], ((0, pad), (0, 0))).T
    dht = jnp.pad(br[:, 1:, 3], ((0, pad), (0, 0))).T
    logits_t = lg.T
    prop_t = jnp.pad(proposals.T, ((0, 4), (0, pad)))

    scm, x1cm, y1cm, x2cm, y2cm = _dense(logits_t, dxt, dyt, dwt, dht, prop_t)
    ks, kx1, ky1, kx2, ky2 = _nms(scm, x1cm, y1cm, x2cm, y2cm)
    ob, osc, olb = _merge(ks, kx1, ky1, kx2, ky2,
                          scm, x1cm, y1cm, x2cm, y2cm)
    det_boxes = ob[: DETS * 4].reshape(DETS, 4)
    return det_boxes, osc[:DETS], olb[:DETS]
